# Initial kernel scaffold; baseline (speedup 1.0000x reference)
import jax
import jax.numpy as jnp
from jax.experimental import pallas as pl


def kernel(x, edge_index, batch, W1, b1, W2, b2, W3, b3):
    raise NotImplementedError("write your pallas kernel here")


# speedup vs baseline: 5.1673x; 5.1673x over previous
import jax
import jax.numpy as jnp
from jax import lax
from jax.experimental import pallas as pl
from jax.experimental.pallas import tpu as pltpu
from jax.experimental.pallas import tpu_sc as plsc
import functools

N, E, D, H, G = 10000, 320000, 128, 128, 64
NC, NS = 2, 16
NW = NC * NS
EPW = E // NW
C = 80
NCHUNK = EPW // C
RPS = 624
TAIL = N - NS * RPS

BLK = 1000
NBLK = N // BLK


def _sc_agg_body(x_hbm, src_hbm, dst_hbm, zeros_hbm, out_hbm,
                 src_v, dst_v, rows_v, sem, agg_sh):
    c = lax.axis_index("c")
    s = lax.axis_index("s")
    wid = c * NS + s

    pltpu.sync_copy(zeros_hbm.at[pl.ds(s * RPS, RPS)],
                    agg_sh.at[pl.ds(s * RPS, RPS)])

    @pl.when(s == NS - 1)
    def _():
        pltpu.sync_copy(zeros_hbm.at[pl.ds(NS * RPS, TAIL)],
                        agg_sh.at[pl.ds(NS * RPS, TAIL)])

    plsc.subcore_barrier()

    e_base = wid * EPW

    def chunk(j, carry):
        base = e_base + j * C
        pltpu.sync_copy(src_hbm.at[pl.ds(base, C)], src_v)
        pltpu.sync_copy(dst_hbm.at[pl.ds(base, C)], dst_v)
        pltpu.async_copy(x_hbm.at[src_v], rows_v, sem).wait()
        pltpu.sync_copy(rows_v, agg_sh.at[dst_v], add=True)
        return carry

    lax.fori_loop(0, NCHUNK, chunk, 0)
    plsc.subcore_barrier()

    pltpu.sync_copy(agg_sh.at[pl.ds(s * RPS, RPS)],
                    out_hbm.at[c, pl.ds(s * RPS, RPS)])

    @pl.when(s == NS - 1)
    def _():
        pltpu.sync_copy(agg_sh.at[pl.ds(NS * RPS, TAIL)],
                        out_hbm.at[c, pl.ds(NS * RPS, TAIL)])


@functools.cache
def _sc_agg():
    return pl.kernel(
        _sc_agg_body,
        out_type=jax.ShapeDtypeStruct((NC, N, D), jnp.float32),
        mesh=plsc.VectorSubcoreMesh(core_axis_name="c", subcore_axis_name="s",
                                    num_cores=NC, num_subcores=NS),
        scratch_types=[
            pltpu.VMEM((C,), jnp.int32),
            pltpu.VMEM((C,), jnp.int32),
            pltpu.VMEM((C, D), jnp.float32),
            pltpu.SemaphoreType.DMA,
            pltpu.VMEM_SHARED((N, D), jnp.float32),
        ],
    )


def _tc_body(x_ref, parts_ref, batch_ref, W1_ref, b1_ref, W2_ref, b2_ref,
             W3_ref, b3_ref, out_ref, pooled_acc):
    i = pl.program_id(0)
    h = x_ref[...] + parts_ref[0] + parts_ref[1]
    h1 = jnp.dot(h, W1_ref[...], preferred_element_type=jnp.float32)
    h1 = jnp.maximum(h1 + b1_ref[...], 0.0)
    h2 = jnp.dot(h1, W2_ref[...], preferred_element_type=jnp.float32)
    h2 = h2 + b2_ref[...]
    bm = batch_ref[0, 0, :]
    gids = lax.broadcasted_iota(jnp.int32, (G, BLK), 0)
    mask = (bm[None, :] == gids).astype(jnp.float32)
    p = jnp.dot(mask, h2, preferred_element_type=jnp.float32)

    @pl.when(i == 0)
    def _():
        pooled_acc[...] = jnp.zeros_like(pooled_acc)

    pooled_acc[...] += p

    @pl.when(i == pl.num_programs(0) - 1)
    def _():
        out_ref[...] = (jnp.dot(pooled_acc[...], W3_ref[...],
                                preferred_element_type=jnp.float32)
                        + b3_ref[...])


@functools.partial(jax.jit)
def _tc_mlp_pool(x, parts, batch3, W1, b1, W2, b2, W3, b3):
    return pl.pallas_call(
        _tc_body,
        grid=(NBLK,),
        in_specs=[
            pl.BlockSpec((BLK, D), lambda i: (i, 0)),
            pl.BlockSpec((NC, BLK, D), lambda i: (0, i, 0)),
            pl.BlockSpec((1, 1, BLK), lambda i: (i, 0, 0)),
            pl.BlockSpec((D, H), lambda i: (0, 0)),
            pl.BlockSpec((1, H), lambda i: (0, 0)),
            pl.BlockSpec((H, H), lambda i: (0, 0)),
            pl.BlockSpec((1, H), lambda i: (0, 0)),
            pl.BlockSpec((H, 1), lambda i: (0, 0)),
            pl.BlockSpec((1, 1), lambda i: (0, 0)),
        ],
        out_specs=pl.BlockSpec((G, 1), lambda i: (0, 0)),
        out_shape=jax.ShapeDtypeStruct((G, 1), jnp.float32),
        scratch_shapes=[pltpu.VMEM((G, H), jnp.float32)],
        compiler_params=pltpu.CompilerParams(
            dimension_semantics=("arbitrary",)),
    )(x, parts, batch3, W1, b1, W2, b2, W3, b3)


def kernel(x, edge_index, batch, W1, b1, W2, b2, W3, b3):
    src = edge_index[0]
    dst = edge_index[1]
    zeros = jnp.zeros_like(x)
    parts = _sc_agg()(x, src, dst, zeros)
    out = _tc_mlp_pool(x, parts, batch.reshape(NBLK, 1, BLK),
                       W1, b1.reshape(1, H), W2, b2.reshape(1, H),
                       W3, b3.reshape(1, 1))
    return out


# speedup vs baseline: 9.1538x; 1.7715x over previous
import jax
import jax.numpy as jnp
from jax import lax
from jax.experimental import pallas as pl
from jax.experimental.pallas import tpu as pltpu
from jax.experimental.pallas import tpu_sc as plsc
import functools

N, E, D, H, G = 10000, 320000, 128, 128, 64
NC, NS = 2, 16
NW = NC * NS
EPW = E // NW
C = 80
NCHUNK = EPW // C
RPS = 624
TAIL = N - NS * RPS

BLK = 1000
NBLK = N // BLK


def _sc_agg_body(x_hbm, src_hbm, dst_hbm, zeros_hbm, out_hbm,
                 src_a, src_b, dst_a, dst_b, rows_a, rows_b,
                 sem_a, sem_b, sem_ia, sem_ib, agg_sh):
    c = lax.axis_index("c")
    s = lax.axis_index("s")
    wid = c * NS + s

    pltpu.sync_copy(zeros_hbm.at[pl.ds(s * RPS, RPS)],
                    agg_sh.at[pl.ds(s * RPS, RPS)])

    @pl.when(s == NS - 1)
    def _():
        pltpu.sync_copy(zeros_hbm.at[pl.ds(NS * RPS, TAIL)],
                        agg_sh.at[pl.ds(NS * RPS, TAIL)])

    plsc.subcore_barrier()
    e_base = wid * EPW

    def idx_fetch(j, sbuf, dbuf, sem):
        base = e_base + j * C
        pltpu.async_copy(src_hbm.at[pl.ds(base, C)], sbuf, sem)
        pltpu.async_copy(dst_hbm.at[pl.ds(base, C)], dbuf, sem)

    def idx_wait(sbuf, dbuf, sem):
        pltpu.make_async_copy(src_hbm.at[pl.ds(0, C)], sbuf, sem).wait()
        pltpu.make_async_copy(dst_hbm.at[pl.ds(0, C)], dbuf, sem).wait()

    def gather(sbuf, buf, sem):
        pltpu.async_copy(x_hbm.at[sbuf], buf, sem)

    def gather_wait(sbuf, buf, sem):
        pltpu.make_async_copy(x_hbm.at[sbuf], buf, sem).wait()

    def scat(buf, dbuf):
        pltpu.sync_copy(buf, agg_sh.at[dbuf], add=True)

    idx_fetch(0, src_a, dst_a, sem_ia)
    idx_wait(src_a, dst_a, sem_ia)
    idx_fetch(1, src_b, dst_b, sem_ib)
    gather(src_a, rows_a, sem_a)

    def body(i, carry):
        ja = 2 * i
        jb = 2 * i + 1
        idx_wait(src_b, dst_b, sem_ib)
        gather_wait(src_a, rows_a, sem_a)
        gather(src_b, rows_b, sem_b)
        scat(rows_a, dst_a)
        idx_fetch(ja + 2, src_a, dst_a, sem_ia)
        idx_wait(src_a, dst_a, sem_ia)
        gather_wait(src_b, rows_b, sem_b)
        gather(src_a, rows_a, sem_a)
        scat(rows_b, dst_b)

        @pl.when(jb + 2 < NCHUNK)
        def _():
            idx_fetch(jb + 2, src_b, dst_b, sem_ib)

        return carry

    lax.fori_loop(0, NCHUNK // 2, body, 0)
    gather_wait(src_a, rows_a, sem_a)
    scat(rows_a, dst_a)
    plsc.subcore_barrier()

    pltpu.sync_copy(agg_sh.at[pl.ds(s * RPS, RPS)],
                    out_hbm.at[c, pl.ds(s * RPS, RPS)])

    @pl.when(s == NS - 1)
    def _():
        pltpu.sync_copy(agg_sh.at[pl.ds(NS * RPS, TAIL)],
                        out_hbm.at[c, pl.ds(NS * RPS, TAIL)])


@functools.cache
def _sc_agg():
    return pl.kernel(
        _sc_agg_body,
        out_type=jax.ShapeDtypeStruct((NC, N, D), jnp.float32),
        mesh=plsc.VectorSubcoreMesh(core_axis_name="c", subcore_axis_name="s",
                                    num_cores=NC, num_subcores=NS),
        scratch_types=[
            pltpu.VMEM((C,), jnp.int32),
            pltpu.VMEM((C,), jnp.int32),
            pltpu.VMEM((C,), jnp.int32),
            pltpu.VMEM((C,), jnp.int32),
            pltpu.VMEM((C, D), jnp.float32),
            pltpu.VMEM((C, D), jnp.float32),
            pltpu.SemaphoreType.DMA,
            pltpu.SemaphoreType.DMA,
            pltpu.SemaphoreType.DMA,
            pltpu.SemaphoreType.DMA,
            pltpu.VMEM_SHARED((N, D), jnp.float32),
        ],
    )


def _tc_body(x_ref, parts_ref, batch_ref, W1_ref, b1_ref, W2_ref, b2_ref,
             W3_ref, b3_ref, out_ref, pooled_acc):
    i = pl.program_id(0)
    h = x_ref[...] + parts_ref[0] + parts_ref[1]
    h1 = jnp.dot(h, W1_ref[...], preferred_element_type=jnp.float32)
    h1 = jnp.maximum(h1 + b1_ref[...], 0.0)
    h2 = jnp.dot(h1, W2_ref[...], preferred_element_type=jnp.float32)
    h2 = h2 + b2_ref[...]
    bm = batch_ref[0, 0, :]
    gids = lax.broadcasted_iota(jnp.int32, (G, BLK), 0)
    mask = (bm[None, :] == gids).astype(jnp.float32)
    p = jnp.dot(mask, h2, preferred_element_type=jnp.float32)

    @pl.when(i == 0)
    def _():
        pooled_acc[...] = jnp.zeros_like(pooled_acc)

    pooled_acc[...] += p

    @pl.when(i == pl.num_programs(0) - 1)
    def _():
        out_ref[...] = (jnp.dot(pooled_acc[...], W3_ref[...],
                                preferred_element_type=jnp.float32)
                        + b3_ref[...])


@functools.partial(jax.jit)
def _tc_mlp_pool(x, parts, batch3, W1, b1, W2, b2, W3, b3):
    return pl.pallas_call(
        _tc_body,
        grid=(NBLK,),
        in_specs=[
            pl.BlockSpec((BLK, D), lambda i: (i, 0)),
            pl.BlockSpec((NC, BLK, D), lambda i: (0, i, 0)),
            pl.BlockSpec((1, 1, BLK), lambda i: (i, 0, 0)),
            pl.BlockSpec((D, H), lambda i: (0, 0)),
            pl.BlockSpec((1, H), lambda i: (0, 0)),
            pl.BlockSpec((H, H), lambda i: (0, 0)),
            pl.BlockSpec((1, H), lambda i: (0, 0)),
            pl.BlockSpec((H, 1), lambda i: (0, 0)),
            pl.BlockSpec((1, 1), lambda i: (0, 0)),
        ],
        out_specs=pl.BlockSpec((G, 1), lambda i: (0, 0)),
        out_shape=jax.ShapeDtypeStruct((G, 1), jnp.float32),
        scratch_shapes=[pltpu.VMEM((G, H), jnp.float32)],
        compiler_params=pltpu.CompilerParams(
            dimension_semantics=("arbitrary",)),
    )(x, parts, batch3, W1, b1, W2, b2, W3, b3)


def kernel(x, edge_index, batch, W1, b1, W2, b2, W3, b3):
    src = edge_index[0]
    dst = edge_index[1]
    zeros = jnp.zeros_like(x)
    parts = _sc_agg()(x, src, dst, zeros)
    out = _tc_mlp_pool(x, parts, batch.reshape(NBLK, 1, BLK),
                       W1, b1.reshape(1, H), W2, b2.reshape(1, H),
                       W3, b3.reshape(1, 1))
    return out
